# baseline (speedup 1.0000x reference)
import jax
import jax.numpy as jnp
from jax import lax
from jax.experimental import pallas as pl
from jax.experimental.pallas import tpu as pltpu
from jax.experimental.pallas import tpu_sc as plsc

N = 10000
E = 320000
D = 128
H = 8
DH = 16
HD = H * DH
ZW = 16
KW = HD + 16
C = 32
NW = 32
N_PAD = 10240
ROWS_PER_TILE = N_PAD // 16
NCHUNKS = E // C


def _proj_body(x_ref, w_ref, k_ref, q_ref, v_ref):
    y = jnp.dot(x_ref[...], w_ref[...], preferred_element_type=jnp.float32)
    q_ref[...] = y[:, :HD]
    k_ref[...] = y[:, HD:2 * HD]
    v_ref[...] = y[:, 2 * HD:]


def _proj(x, w, bm):
    m, k = x.shape
    return pl.pallas_call(
        _proj_body,
        grid=(m // bm,),
        in_specs=[pl.BlockSpec((bm, k), lambda i: (i, 0)),
                  pl.BlockSpec((k, 3 * HD), lambda i: (0, 0))],
        out_specs=[pl.BlockSpec((bm, HD), lambda i: (i, 0)),
                   pl.BlockSpec((bm, HD), lambda i: (i, 0)),
                   pl.BlockSpec((bm, HD), lambda i: (i, 0))],
        out_shape=[jax.ShapeDtypeStruct((m, HD), jnp.float32),
                   jax.ShapeDtypeStruct((m, HD), jnp.float32),
                   jax.ShapeDtypeStruct((m, HD), jnp.float32)],
    )(x, w)


def _split_body(e_ref, r_ref, c_ref):
    r_ref[...] = e_ref[0, :]
    c_ref[...] = e_ref[1, :]


def _split(ei, bn):
    return pl.pallas_call(
        _split_body,
        grid=(E // bn,),
        in_specs=[pl.BlockSpec((2, bn), lambda i: (0, i))],
        out_specs=[pl.BlockSpec((bn,), lambda i: (i,)),
                   pl.BlockSpec((bn,), lambda i: (i,))],
        out_shape=[jax.ShapeDtypeStruct((E,), jnp.int32),
                   jax.ShapeDtypeStruct((E,), jnp.int32)],
    )(ei)


def _mm_body(a_ref, w_ref, o_ref):
    o_ref[...] = jnp.dot(a_ref[...], w_ref[...],
                         preferred_element_type=jnp.float32)


def _mm(a, w, bm):
    m, k = a.shape
    _, n = w.shape
    return pl.pallas_call(
        _mm_body,
        grid=(m // bm,),
        in_specs=[pl.BlockSpec((bm, k), lambda i: (i, 0)),
                  pl.BlockSpec((k, n), lambda i: (0, 0))],
        out_specs=pl.BlockSpec((bm, n), lambda i: (i, 0)),
        out_shape=jax.ShapeDtypeStruct((m, n), jnp.float32),
    )(a, w)


def _combine_body(p0w_ref, p1w_ref, p0z_ref, p1z_ref, ex_ref, o_ref):
    wv = p0w_ref[...] + p1w_ref[...]
    z = p0z_ref[...] + p1z_ref[...]
    zb = jnp.dot(z, ex_ref[...], preferred_element_type=jnp.float32)
    o_ref[...] = wv / (zb + 1e-6)


def _combine(p0w, p1w, p0z, p1z, ex, bm):
    return pl.pallas_call(
        _combine_body,
        grid=(N_PAD // bm,),
        in_specs=[pl.BlockSpec((bm, HD), lambda i: (i, 0)),
                  pl.BlockSpec((bm, HD), lambda i: (i, 0)),
                  pl.BlockSpec((bm, ZW), lambda i: (i, 0)),
                  pl.BlockSpec((bm, ZW), lambda i: (i, 0)),
                  pl.BlockSpec((ZW, HD), lambda i: (0, 0))],
        out_specs=pl.BlockSpec((bm, HD), lambda i: (i, 0)),
        out_shape=jax.ShapeDtypeStruct((N_PAD, HD), jnp.float32),
    )(p0w, p1w, p0z, p1z, ex)


def _edge_kernel(kc_hbm, qc_hbm, v_hbm, cp_hbm, epb_hbm, row_hbm, col_hbm,
                 we_hbm, zw_hbm, zz_hbm, eout_hbm, partw_hbm, partz_hbm,
                 rowi0, rowi1, coli0, coli1, colw0, colw1,
                 kcb0, kcb1, qcb0, qcb1, crb0, crb1, ccb0, ccb1,
                 vb0, vb1, epb0, epb1, zb0, zb1, wec,
                 accw, accz, semi0, semi1, semg0, semg1, semw0, semw1):
    cid = lax.axis_index("c")
    sid = lax.axis_index("s")
    wid = sid * 2 + cid
    rowi = (rowi0, rowi1)
    coli = (coli0, coli1)
    colw = (colw0, colw1)
    kcb = (kcb0, kcb1)
    qcb = (qcb0, qcb1)
    crb = (crb0, crb1)
    ccb = (ccb0, ccb1)
    vb = (vb0, vb1)
    epb = (epb0, epb1)
    zb = (zb0, zb1)
    semi = (semi0, semi1)
    semg = (semg0, semg1)
    semw = (semw0, semw1)

    pltpu.sync_copy(we_hbm, wec)
    acc_off = pl.multiple_of(sid * ROWS_PER_TILE, 8)
    pltpu.sync_copy(zw_hbm, accw.at[pl.ds(acc_off, ROWS_PER_TILE)])
    pltpu.sync_copy(zz_hbm, accz.at[pl.ds(acc_off, ROWS_PER_TILE)])
    plsc.subcore_barrier()

    nloc = jnp.where(wid < (NCHUNKS % NW), NCHUNKS // NW + 1, NCHUNKS // NW)
    lanes = lax.iota(jnp.int32, 16)

    def cbase(j):
        return pl.multiple_of((wid + NW * j) * C, C)

    def issue_idx(j, s):
        b = cbase(j)
        pltpu.async_copy(row_hbm.at[pl.ds(b, C)], rowi[s], semi[s])
        pltpu.async_copy(col_hbm.at[pl.ds(b, C)], coli[s], semi[s])

    def wait_idx(j, s):
        b = cbase(j)
        pltpu.make_async_copy(row_hbm.at[pl.ds(b, C)], rowi[s],
                              semi[s]).wait()
        pltpu.make_async_copy(col_hbm.at[pl.ds(b, C)], coli[s],
                              semi[s]).wait()

    def issue_gathers(j, s):
        b = cbase(j)
        pltpu.async_copy(kc_hbm.at[rowi[s]], kcb[s], semg[s])
        pltpu.async_copy(qc_hbm.at[coli[s]], qcb[s], semg[s])
        pltpu.async_copy(v_hbm.at[rowi[s]], vb[s], semg[s])
        pltpu.async_copy(cp_hbm.at[rowi[s]], crb[s], semg[s])
        pltpu.async_copy(cp_hbm.at[coli[s]], ccb[s], semg[s])
        pltpu.async_copy(epb_hbm.at[pl.ds(b, C)], epb[s], semg[s])

    def wait_gathers(j, s):
        b = cbase(j)
        pltpu.make_async_copy(kc_hbm.at[rowi[s]], kcb[s], semg[s]).wait()
        pltpu.make_async_copy(qc_hbm.at[coli[s]], qcb[s], semg[s]).wait()
        pltpu.make_async_copy(v_hbm.at[rowi[s]], vb[s], semg[s]).wait()
        pltpu.make_async_copy(cp_hbm.at[rowi[s]], crb[s], semg[s]).wait()
        pltpu.make_async_copy(cp_hbm.at[coli[s]], ccb[s], semg[s]).wait()
        pltpu.make_async_copy(epb_hbm.at[pl.ds(b, C)], epb[s],
                              semg[s]).wait()

    def allsum(x):
        cs = jnp.cumsum(x)
        rs = jnp.flip(jnp.cumsum(jnp.flip(x, 0)), 0)
        return cs + rs - x

    def issue_writes(j, s):
        b = cbase(j)
        pltpu.async_copy(epb[s], eout_hbm.at[pl.ds(b, C)], semw[s])
        pltpu.sync_copy(vb[s], accw.at[coli[s]], add=True)
        pltpu.sync_copy(zb[s], accz.at[coli[s]], add=True)

    def wait_writes(s):
        pltpu.make_async_copy(epb[s], eout_hbm.at[pl.ds(0, C)],
                              semw[s]).wait()

    def compute(j, s):
        kcr, qcr, vr, epr = kcb[s], qcb[s], vb[s], epb[s]
        zbuf = zb[s]

        def edge_body(e):
            dvec = crb[s][e, :] - ccb[s][e, :]
            sv = allsum(dvec * dvec)
            ii = plsc.bitcast(sv, jnp.int32)
            ii = 0x5F3759DF - (ii >> 1)
            y = plsc.bitcast(ii, jnp.float32)
            hs = sv * 0.5
            for _ in range(3):
                t = hs * y
                t = t * y
                y = y * (1.5 - t)
            dist = sv * y * 0.1

            zacc = jnp.zeros((16,), jnp.float32)
            for h in range(H):
                sl = pl.ds(h * DH, DH)
                kq = jnp.clip(kcr[e, sl] * qcr[e, sl], -5.0, 5.0)
                al = kq * (epr[e, sl] + dist * wec[h, :])
                epr[e, sl] = al
                ax = jnp.exp(jnp.clip(allsum(al), -5.0, 5.0))
                vr[e, sl] = vr[e, sl] * ax
                zacc = zacc + jnp.where(lanes == h, ax, 0.0)
            zbuf[e, :] = zacc

        plsc.parallel_loop(0, C, unroll=2)(edge_body)
        issue_writes(j, s)

    b0 = cbase(0)
    pltpu.sync_copy(row_hbm.at[pl.ds(b0, C)], rowi[0])
    pltpu.sync_copy(col_hbm.at[pl.ds(b0, C)], coli[0])
    issue_gathers(0, 0)

    @pl.when(nloc > 1)
    def _():
        issue_idx(1, 1)

    def outer_body(jj, carry):
        for s in (0, 1):
            j = 2 * jj + s
            q = 1 - s

            @pl.when(j < nloc)
            def _():
                @pl.when(j + 1 < nloc)
                def _():
                    @pl.when(j >= 1)
                    def _():
                        wait_writes(q)

                    wait_idx(j + 1, q)
                    issue_gathers(j + 1, q)

                wait_gathers(j, s)
                compute(j, s)

                @pl.when(j + 2 < nloc)
                def _():
                    issue_idx(j + 2, s)
        return carry

    lax.fori_loop(0, (nloc + 1) // 2, outer_body, 0)

    wait_writes(0)

    @pl.when(nloc >= 2)
    def _():
        wait_writes(1)

    plsc.subcore_barrier()
    pltpu.sync_copy(accw.at[pl.ds(acc_off, ROWS_PER_TILE)],
                    partw_hbm.at[cid, pl.ds(acc_off, ROWS_PER_TILE)])
    pltpu.sync_copy(accz.at[pl.ds(acc_off, ROWS_PER_TILE)],
                    partz_hbm.at[cid, pl.ds(acc_off, ROWS_PER_TILE)])


_edge_call = pl.kernel(
    mesh=plsc.VectorSubcoreMesh(core_axis_name="c", subcore_axis_name="s"),
    compiler_params=pltpu.CompilerParams(needs_layout_passes=False,
                                         use_tc_tiling_on_sc=False),
    out_type=[jax.ShapeDtypeStruct((E, HD), jnp.float32),
              jax.ShapeDtypeStruct((2, N_PAD, HD), jnp.float32),
              jax.ShapeDtypeStruct((2, N_PAD, ZW), jnp.float32)],
    scratch_types=[
        pltpu.VMEM((C,), jnp.int32),
        pltpu.VMEM((C,), jnp.int32),
        pltpu.VMEM((C,), jnp.int32),
        pltpu.VMEM((C,), jnp.int32),
        pltpu.VMEM((C,), jnp.int32),
        pltpu.VMEM((C,), jnp.int32),
        pltpu.VMEM((C, HD), jnp.float32),
        pltpu.VMEM((C, HD), jnp.float32),
        pltpu.VMEM((C, HD), jnp.float32),
        pltpu.VMEM((C, HD), jnp.float32),
        pltpu.VMEM((C, 16), jnp.float32),
        pltpu.VMEM((C, 16), jnp.float32),
        pltpu.VMEM((C, 16), jnp.float32),
        pltpu.VMEM((C, 16), jnp.float32),
        pltpu.VMEM((C, HD), jnp.float32),
        pltpu.VMEM((C, HD), jnp.float32),
        pltpu.VMEM((C, HD), jnp.float32),
        pltpu.VMEM((C, HD), jnp.float32),
        pltpu.VMEM((C, ZW), jnp.float32),
        pltpu.VMEM((C, ZW), jnp.float32),
        pltpu.VMEM((H, DH), jnp.float32),
        pltpu.VMEM_SHARED((N_PAD, HD), jnp.float32),
        pltpu.VMEM_SHARED((N_PAD, ZW), jnp.float32),
        pltpu.SemaphoreType.DMA,
        pltpu.SemaphoreType.DMA,
        pltpu.SemaphoreType.DMA,
        pltpu.SemaphoreType.DMA,
        pltpu.SemaphoreType.DMA,
        pltpu.SemaphoreType.DMA,
    ],
)(_edge_kernel)


def kernel(x, edge_attr, edge_index, coords, WQ, WK, WV, WE):
    scale = 1.0 / (DH ** 0.5)
    wcat = jnp.concatenate([WQ, WK * scale, WV], axis=1)
    cpad = jnp.pad(coords, ((0, 0), (0, 16 - coords.shape[1])))
    kc, qc, v = _proj(x, wcat, 400)
    epb = _mm(edge_attr, WE[:D], 512)

    we_last = WE[D].reshape(H, DH)
    zerosw = jnp.zeros((ROWS_PER_TILE, HD), jnp.float32)
    zerosz = jnp.zeros((ROWS_PER_TILE, ZW), jnp.float32)

    row, col = _split(edge_index, 512)
    e_out, partw, partz = _edge_call(kc, qc, v, cpad, epb, row, col,
                                     we_last, zerosw, zerosz)

    ex = (jnp.arange(ZW)[:, None] == (jnp.arange(HD)[None, :] // DH)
          ).astype(jnp.float32)
    h_out = _combine(partw[0], partw[1], partz[0], partz[1], ex, 640)[:N]

    return (h_out.reshape(N, H, DH), e_out.reshape(E, H, DH), coords)


# speedup vs baseline: 1.1730x; 1.1730x over previous
import jax
import jax.numpy as jnp
from jax import lax
from jax.experimental import pallas as pl
from jax.experimental.pallas import tpu as pltpu
from jax.experimental.pallas import tpu_sc as plsc

N = 10000
E = 320000
D = 128
H = 8
DH = 16
HD = H * DH
ZW = 16
KW = HD + 16
C = 32
NW = 32
N_PAD = 10240
ROWS_PER_TILE = N_PAD // 16
NCHUNKS = E // C


def _proj_body(x_ref, w_ref, k_ref, q_ref, v_ref):
    y = jnp.dot(x_ref[...], w_ref[...], preferred_element_type=jnp.float32)
    q_ref[...] = y[:, :HD]
    k_ref[...] = y[:, HD:2 * HD]
    v_ref[...] = y[:, 2 * HD:]


def _proj(x, w, bm):
    m, k = x.shape
    return pl.pallas_call(
        _proj_body,
        grid=(m // bm,),
        in_specs=[pl.BlockSpec((bm, k), lambda i: (i, 0)),
                  pl.BlockSpec((k, 3 * HD), lambda i: (0, 0))],
        out_specs=[pl.BlockSpec((bm, HD), lambda i: (i, 0)),
                   pl.BlockSpec((bm, HD), lambda i: (i, 0)),
                   pl.BlockSpec((bm, HD), lambda i: (i, 0))],
        out_shape=[jax.ShapeDtypeStruct((m, HD), jnp.float32),
                   jax.ShapeDtypeStruct((m, HD), jnp.float32),
                   jax.ShapeDtypeStruct((m, HD), jnp.float32)],
    )(x, w)


def _mm_body(a_ref, w_ref, o_ref):
    o_ref[...] = jnp.dot(a_ref[...], w_ref[...],
                         preferred_element_type=jnp.float32)


def _mm(a, w, bm):
    m, k = a.shape
    _, n = w.shape
    return pl.pallas_call(
        _mm_body,
        grid=(m // bm,),
        in_specs=[pl.BlockSpec((bm, k), lambda i: (i, 0)),
                  pl.BlockSpec((k, n), lambda i: (0, 0))],
        out_specs=pl.BlockSpec((bm, n), lambda i: (i, 0)),
        out_shape=jax.ShapeDtypeStruct((m, n), jnp.float32),
    )(a, w)


def _combine_body(p0w_ref, p1w_ref, p0z_ref, p1z_ref, ex_ref, o_ref):
    wv = p0w_ref[...] + p1w_ref[...]
    z = p0z_ref[...] + p1z_ref[...]
    zb = jnp.dot(z, ex_ref[...], preferred_element_type=jnp.float32)
    o_ref[...] = wv / (zb + 1e-6)


def _combine(p0w, p1w, p0z, p1z, ex, bm):
    return pl.pallas_call(
        _combine_body,
        grid=(N_PAD // bm,),
        in_specs=[pl.BlockSpec((bm, HD), lambda i: (i, 0)),
                  pl.BlockSpec((bm, HD), lambda i: (i, 0)),
                  pl.BlockSpec((bm, ZW), lambda i: (i, 0)),
                  pl.BlockSpec((bm, ZW), lambda i: (i, 0)),
                  pl.BlockSpec((ZW, HD), lambda i: (0, 0))],
        out_specs=pl.BlockSpec((bm, HD), lambda i: (i, 0)),
        out_shape=jax.ShapeDtypeStruct((N_PAD, HD), jnp.float32),
    )(p0w, p1w, p0z, p1z, ex)


def _edge_kernel(kc_hbm, qc_hbm, v_hbm, cp_hbm, epb_hbm, ei_hbm,
                 we_hbm, zw_hbm, zz_hbm, eout_hbm, partw_hbm, partz_hbm,
                 rowi0, rowi1, coli0, coli1, colw0, colw1,
                 kcb0, kcb1, qcb0, qcb1, crb0, crb1, ccb0, ccb1,
                 vb0, vb1, epb0, epb1, zb0, zb1, wec,
                 accw, accz, semi0, semi1, semg0, semg1, semw0, semw1):
    cid = lax.axis_index("c")
    sid = lax.axis_index("s")
    wid = sid * 2 + cid
    rowi = (rowi0, rowi1)
    coli = (coli0, coli1)
    colw = (colw0, colw1)
    kcb = (kcb0, kcb1)
    qcb = (qcb0, qcb1)
    crb = (crb0, crb1)
    ccb = (ccb0, ccb1)
    vb = (vb0, vb1)
    epb = (epb0, epb1)
    zb = (zb0, zb1)
    semi = (semi0, semi1)
    semg = (semg0, semg1)
    semw = (semw0, semw1)

    pltpu.sync_copy(we_hbm, wec)
    acc_off = pl.multiple_of(sid * ROWS_PER_TILE, 8)
    pltpu.sync_copy(zw_hbm, accw.at[pl.ds(acc_off, ROWS_PER_TILE)])
    pltpu.sync_copy(zz_hbm, accz.at[pl.ds(acc_off, ROWS_PER_TILE)])
    plsc.subcore_barrier()

    nloc = jnp.where(wid < (NCHUNKS % NW), NCHUNKS // NW + 1, NCHUNKS // NW)
    lanes = lax.iota(jnp.int32, 16)

    def cbase(j):
        return pl.multiple_of((wid + NW * j) * C, C)

    def issue_idx(j, s):
        b = cbase(j)
        pltpu.async_copy(ei_hbm.at[0, pl.ds(b, C)], rowi[s], semi[s])
        pltpu.async_copy(ei_hbm.at[1, pl.ds(b, C)], coli[s], semi[s])

    def wait_idx(j, s):
        b = cbase(j)
        pltpu.make_async_copy(ei_hbm.at[0, pl.ds(b, C)], rowi[s],
                              semi[s]).wait()
        pltpu.make_async_copy(ei_hbm.at[1, pl.ds(b, C)], coli[s],
                              semi[s]).wait()

    def issue_gathers(j, s):
        b = cbase(j)
        pltpu.async_copy(kc_hbm.at[rowi[s]], kcb[s], semg[s])
        pltpu.async_copy(qc_hbm.at[coli[s]], qcb[s], semg[s])
        pltpu.async_copy(v_hbm.at[rowi[s]], vb[s], semg[s])
        pltpu.async_copy(cp_hbm.at[rowi[s]], crb[s], semg[s])
        pltpu.async_copy(cp_hbm.at[coli[s]], ccb[s], semg[s])
        pltpu.async_copy(epb_hbm.at[pl.ds(b, C)], epb[s], semg[s])

    def wait_gathers(j, s):
        b = cbase(j)
        pltpu.make_async_copy(kc_hbm.at[rowi[s]], kcb[s], semg[s]).wait()
        pltpu.make_async_copy(qc_hbm.at[coli[s]], qcb[s], semg[s]).wait()
        pltpu.make_async_copy(v_hbm.at[rowi[s]], vb[s], semg[s]).wait()
        pltpu.make_async_copy(cp_hbm.at[rowi[s]], crb[s], semg[s]).wait()
        pltpu.make_async_copy(cp_hbm.at[coli[s]], ccb[s], semg[s]).wait()
        pltpu.make_async_copy(epb_hbm.at[pl.ds(b, C)], epb[s],
                              semg[s]).wait()

    def allsum(x):
        cs = jnp.cumsum(x)
        rs = jnp.flip(jnp.cumsum(jnp.flip(x, 0)), 0)
        return cs + rs - x

    def issue_writes(j, s):
        b = cbase(j)
        pltpu.async_copy(epb[s], eout_hbm.at[pl.ds(b, C)], semw[s])
        pltpu.sync_copy(vb[s], accw.at[coli[s]], add=True)
        pltpu.sync_copy(zb[s], accz.at[coli[s]], add=True)

    def wait_writes(s):
        pltpu.make_async_copy(epb[s], eout_hbm.at[pl.ds(0, C)],
                              semw[s]).wait()

    def compute(j, s):
        kcr, qcr, vr, epr = kcb[s], qcb[s], vb[s], epb[s]
        zbuf = zb[s]

        def edge_body(e):
            dvec = crb[s][e, :] - ccb[s][e, :]
            sv = allsum(dvec * dvec)
            ii = plsc.bitcast(sv, jnp.int32)
            ii = 0x5F3759DF - (ii >> 1)
            y = plsc.bitcast(ii, jnp.float32)
            hs = sv * 0.5
            for _ in range(3):
                t = hs * y
                t = t * y
                y = y * (1.5 - t)
            dist = sv * y * 0.1

            zacc = jnp.zeros((16,), jnp.float32)
            for h in range(H):
                sl = pl.ds(h * DH, DH)
                kq = jnp.clip(kcr[e, sl] * qcr[e, sl], -5.0, 5.0)
                al = kq * (epr[e, sl] + dist * wec[h, :])
                epr[e, sl] = al
                ax = jnp.exp(jnp.clip(allsum(al), -5.0, 5.0))
                vr[e, sl] = vr[e, sl] * ax
                zacc = zacc + jnp.where(lanes == h, ax, 0.0)
            zbuf[e, :] = zacc

        plsc.parallel_loop(0, C, unroll=2)(edge_body)
        issue_writes(j, s)

    b0 = cbase(0)
    pltpu.sync_copy(ei_hbm.at[0, pl.ds(b0, C)], rowi[0])
    pltpu.sync_copy(ei_hbm.at[1, pl.ds(b0, C)], coli[0])
    issue_gathers(0, 0)

    @pl.when(nloc > 1)
    def _():
        issue_idx(1, 1)

    def outer_body(jj, carry):
        for s in (0, 1):
            j = 2 * jj + s
            q = 1 - s

            @pl.when(j < nloc)
            def _():
                @pl.when(j + 1 < nloc)
                def _():
                    @pl.when(j >= 1)
                    def _():
                        wait_writes(q)

                    wait_idx(j + 1, q)
                    issue_gathers(j + 1, q)

                wait_gathers(j, s)
                compute(j, s)

                @pl.when(j + 2 < nloc)
                def _():
                    issue_idx(j + 2, s)
        return carry

    lax.fori_loop(0, (nloc + 1) // 2, outer_body, 0)

    wait_writes(0)

    @pl.when(nloc >= 2)
    def _():
        wait_writes(1)

    plsc.subcore_barrier()
    pltpu.sync_copy(accw.at[pl.ds(acc_off, ROWS_PER_TILE)],
                    partw_hbm.at[cid, pl.ds(acc_off, ROWS_PER_TILE)])
    pltpu.sync_copy(accz.at[pl.ds(acc_off, ROWS_PER_TILE)],
                    partz_hbm.at[cid, pl.ds(acc_off, ROWS_PER_TILE)])


_edge_call = pl.kernel(
    mesh=plsc.VectorSubcoreMesh(core_axis_name="c", subcore_axis_name="s"),
    compiler_params=pltpu.CompilerParams(needs_layout_passes=False,
                                         use_tc_tiling_on_sc=False),
    out_type=[jax.ShapeDtypeStruct((E, HD), jnp.float32),
              jax.ShapeDtypeStruct((2, N_PAD, HD), jnp.float32),
              jax.ShapeDtypeStruct((2, N_PAD, ZW), jnp.float32)],
    scratch_types=[
        pltpu.VMEM((C,), jnp.int32),
        pltpu.VMEM((C,), jnp.int32),
        pltpu.VMEM((C,), jnp.int32),
        pltpu.VMEM((C,), jnp.int32),
        pltpu.VMEM((C,), jnp.int32),
        pltpu.VMEM((C,), jnp.int32),
        pltpu.VMEM((C, HD), jnp.float32),
        pltpu.VMEM((C, HD), jnp.float32),
        pltpu.VMEM((C, HD), jnp.float32),
        pltpu.VMEM((C, HD), jnp.float32),
        pltpu.VMEM((C, 16), jnp.float32),
        pltpu.VMEM((C, 16), jnp.float32),
        pltpu.VMEM((C, 16), jnp.float32),
        pltpu.VMEM((C, 16), jnp.float32),
        pltpu.VMEM((C, HD), jnp.float32),
        pltpu.VMEM((C, HD), jnp.float32),
        pltpu.VMEM((C, HD), jnp.float32),
        pltpu.VMEM((C, HD), jnp.float32),
        pltpu.VMEM((C, ZW), jnp.float32),
        pltpu.VMEM((C, ZW), jnp.float32),
        pltpu.VMEM((H, DH), jnp.float32),
        pltpu.VMEM_SHARED((N_PAD, HD), jnp.float32),
        pltpu.VMEM_SHARED((N_PAD, ZW), jnp.float32),
        pltpu.SemaphoreType.DMA,
        pltpu.SemaphoreType.DMA,
        pltpu.SemaphoreType.DMA,
        pltpu.SemaphoreType.DMA,
        pltpu.SemaphoreType.DMA,
        pltpu.SemaphoreType.DMA,
    ],
)(_edge_kernel)


def kernel(x, edge_attr, edge_index, coords, WQ, WK, WV, WE):
    scale = 1.0 / (DH ** 0.5)
    wcat = jnp.concatenate([WQ, WK * scale, WV], axis=1)
    cpad = jnp.pad(coords, ((0, 0), (0, 16 - coords.shape[1])))
    kc, qc, v = _proj(x, wcat, 400)
    epb = _mm(edge_attr, WE[:D], 512)

    we_last = WE[D].reshape(H, DH)
    zerosw = jnp.zeros((ROWS_PER_TILE, HD), jnp.float32)
    zerosz = jnp.zeros((ROWS_PER_TILE, ZW), jnp.float32)

    e_out, partw, partz = _edge_call(kc, qc, v, cpad, epb, edge_index,
                                     we_last, zerosw, zerosz)

    ex = (jnp.arange(ZW)[:, None] == (jnp.arange(HD)[None, :] // DH)
          ).astype(jnp.float32)
    h_out = _combine(partw[0], partw[1], partz[0], partz[1], ex, 640)[:N]

    return (h_out.reshape(N, H, DH), e_out.reshape(E, H, DH), coords)
